# baseline (speedup 1.0000x reference)
import functools

import jax
import jax.numpy as jnp
from jax import lax
from jax.experimental import pallas as pl
from jax.experimental.pallas import tpu as pltpu
from jax.experimental.pallas import tpu_sc as plsc

_NUM_CORES = 2
_NUM_SUBCORES = 16
_NUM_WORKERS = _NUM_CORES * _NUM_SUBCORES
_L = 16

_WIN = 256
_NBUF = 3
_WL_CAP = 4096
_RING = 128
_CHUNK = 64
_OSENT = 127


def _row_col(pos):
  return [lax.shift_right_logical(pos, 7), pos & 127]


def _make_sweep(batch: int, dim: int, vocab: int):
  n_win = vocab // _WIN
  tail_owner = n_win % _NUM_WORKERS
  wdim = 2 * dim
  out_rows = batch + _NUM_WORKERS
  mesh = plsc.VectorSubcoreMesh(core_axis_name="c", subcore_axis_name="s")

  @functools.partial(
      pl.kernel,
      mesh=mesh,
      out_type=jax.ShapeDtypeStruct((out_rows, wdim), jnp.float32),
      compiler_params=pltpu.CompilerParams(needs_layout_passes=False),
      scratch_types=[
          pltpu.VMEM((batch + _L,), jnp.int32),
          pltpu.VMEM((batch // 128 + 1, 128), jnp.int32),
          pltpu.VMEM((_WL_CAP // 128 + 1, 128), jnp.int32),
          pltpu.VMEM((_NBUF * dim, _WIN), jnp.float32),
          pltpu.VMEM((_RING, wdim), jnp.float32),
          pltpu.VMEM((2, _CHUNK), jnp.int32),
          pltpu.SemaphoreType.DMA,
      ],
  )
  def sweep(idx_hbm, tablet_hbm, tail_hbm, out_hbm,
            idx_v, mk_v, wl_v, win_v, ring_v, blist_v, wsem):
    wid = lax.axis_index("s") * _NUM_CORES + lax.axis_index("c")
    trash = batch + wid
    iota = lax.iota(jnp.int32, _L)
    n_my_win = (n_win - 1 - wid) // _NUM_WORKERS + 1

    pltpu.sync_copy(idx_hbm, idx_v.at[pl.ds(0, batch)])

    for q in range(2 * _CHUNK // _L):
      blist_v[q // (_CHUNK // _L), pl.ds((q % (_CHUNK // _L)) * _L, _L)] = (
          jnp.zeros((_L,), jnp.int32) + trash)

    def scan_body(g, m):
      v = idx_v[pl.ds(g * _L, _L)]
      c = lax.shift_right_logical(v, 8)
      mask = (c & (_NUM_WORKERS - 1)) == wid
      key = (lax.shift_left(lax.shift_right_logical(c, 5), 24)
             | lax.shift_left(v & (_WIN - 1), 15)
             | (iota + g * _L))
      pref = plsc.cumsum(mask.astype(jnp.int32))
      pos = jnp.where(mask, m + pref - 1, batch)
      plsc.store_scatter(mk_v, _row_col(pos), key)
      return m + plsc.all_reduce_population_count(mask)[0]

    m_cnt = lax.fori_loop(0, batch // _L, scan_body, 0)
    plsc.store_scatter(mk_v, _row_col(m_cnt + iota),
                       jnp.zeros((_L,), jnp.int32) + (_OSENT << 24))

    def do_window(boff, k, rpos, spos):

      def blk_body(blk, carry):
        base = blk * _WL_CAP

        def wl_body(g, w):
          kk = plsc.load_gather(mk_v, _row_col(base + g * _L + iota))
          mask = lax.shift_right_logical(kk, 24) == k
          pref = plsc.cumsum(mask.astype(jnp.int32))
          pos = jnp.where(mask, w + pref - 1, _WL_CAP)
          plsc.store_scatter(wl_v, _row_col(pos), kk)
          return w + plsc.all_reduce_population_count(mask)[0]

        bcnt = jnp.minimum(m_cnt - base, _WL_CAP)
        w_cnt = lax.fori_loop(0, (bcnt + _L - 1) // _L, wl_body, 0)
        plsc.store_scatter(wl_v, _row_col(w_cnt + iota),
                           jnp.zeros((_L,), jnp.int32) + trash)

        def grp_body(g2, carry):
          rpos, spos = carry
          kk = plsc.load_gather(wl_v, _row_col(g2 * _L + iota))
          colv = lax.shift_right_logical(kk, 15) & (_WIN - 1)
          bv = kk & 32767
          gcnt = jnp.minimum(w_cnt - g2 * _L, _L)
          rowpos = (rpos + iota) & (_RING - 1)
          for j in range(dim):
            jsplat = jnp.zeros((_L,), jnp.int32) + j
            val = plsc.load_gather(win_v, [jsplat + boff, colv])
            plsc.store_scatter(ring_v, [rowpos, jsplat], val)
          plsc.store_scatter(
              blist_v,
              [lax.shift_right_logical(rowpos, 6), rowpos & (_CHUNK - 1)], bv)
          rpos = rpos + gcnt

          def flush(spos):
            cid = (spos // _CHUNK) & 1
            pltpu.sync_copy(
                ring_v.at[pl.ds(pl.multiple_of(cid * _CHUNK, _CHUNK), _CHUNK)],
                out_hbm.at[blist_v.at[cid]])
            return spos + _CHUNK

          spos = lax.cond(rpos - spos >= _CHUNK, flush, lambda s: s, spos)
          return (rpos, spos)

        return lax.fori_loop(0, (w_cnt + _L - 1) // _L, grp_body, carry)

      n_blk = (m_cnt + _WL_CAP - 1) // _WL_CAP
      return lax.fori_loop(0, n_blk, blk_body, (rpos, spos))

    def fetch(k):
      c = wid + k * _NUM_WORKERS
      off = pl.multiple_of(c * _WIN, _WIN)
      boff = (k % _NBUF) * dim
      return pltpu.async_copy(
          tablet_hbm.at[:, pl.ds(off, _WIN)],
          win_v.at[pl.ds(pl.multiple_of(boff, dim), dim)], wsem)

    for p in range(_NBUF - 1):
      @pl.when(p < n_my_win)
      def _(p=p):
        fetch(p)

    def win_body(k, carry):
      @pl.when(k + _NBUF - 1 < n_my_win)
      def _():
        fetch(k + _NBUF - 1)

      pltpu.make_async_copy(
          tablet_hbm.at[:, pl.ds(0, _WIN)],
          win_v.at[pl.ds(0, dim)], wsem).wait()
      rpos, spos = do_window((k % _NBUF) * dim, k, *carry)
      return (rpos, spos)

    rpos, spos = lax.fori_loop(0, n_my_win, win_body, (0, 0))

    def tail_fn(carry):
      pltpu.sync_copy(tail_hbm, win_v.at[pl.ds(0, dim), pl.ds(0, 128)])
      rpos, spos = do_window(0, n_my_win, *carry)
      return (rpos, spos)

    rpos, spos = lax.cond(
        wid == tail_owner, tail_fn, lambda carry: carry, (rpos, spos))

    for cid in range(2):
      pltpu.sync_copy(
          ring_v.at[pl.ds(cid * _CHUNK, _CHUNK)],
          out_hbm.at[blist_v.at[cid]])

  return sweep


def kernel(idx, text_embeds):
  vocab, dim = text_embeds.shape
  (batch,) = idx.shape
  tablet = text_embeds.T
  tail_lo = (vocab // _WIN) * _WIN
  tail = jnp.pad(tablet[:, tail_lo:], ((0, 0), (0, 128 - (vocab - tail_lo))))
  out_p = _make_sweep(batch, dim, vocab)(idx.astype(jnp.int32), tablet, tail)
  return out_p[:batch, :dim]


# speedup vs baseline: 1.2593x; 1.2593x over previous
import functools

import jax
import jax.numpy as jnp
from jax import lax
from jax.experimental import pallas as pl
from jax.experimental.pallas import tpu as pltpu
from jax.experimental.pallas import tpu_sc as plsc

_NUM_CORES = 2
_NUM_SUBCORES = 16
_NUM_WORKERS = _NUM_CORES * _NUM_SUBCORES
_L = 16

_WIN = 512
_WL_CAP = 4096
_RING = 128
_CHUNK = 64
_OSENT = 63


def _row_col(pos):
  return [lax.shift_right_logical(pos, 7), pos & 127]


def _make_sweep(batch: int, dim: int, vocab: int):
  n_win = vocab // _WIN
  tail_owner = n_win % _NUM_WORKERS
  wdim = 2 * dim
  out_rows = batch + _NUM_WORKERS
  mesh = plsc.VectorSubcoreMesh(core_axis_name="c", subcore_axis_name="s")

  @functools.partial(
      pl.kernel,
      mesh=mesh,
      out_type=jax.ShapeDtypeStruct((out_rows, wdim), jnp.float32),
      compiler_params=pltpu.CompilerParams(needs_layout_passes=False),
      scratch_types=[
          pltpu.VMEM((batch + _L,), jnp.int32),
          pltpu.VMEM((batch // 128 + 1, 128), jnp.int32),
          pltpu.VMEM((_WL_CAP // 128 + 1, 128), jnp.int32),
          pltpu.VMEM((2 * dim, _WIN), jnp.float32),
          pltpu.VMEM((_RING, wdim), jnp.float32),
          pltpu.VMEM((2, _CHUNK), jnp.int32),
          pltpu.SemaphoreType.DMA,
      ],
  )
  def sweep(idx_hbm, tablet_hbm, tail_hbm, out_hbm,
            idx_v, mk_v, wl_v, win_v, ring_v, blist_v, wsem):
    wid = lax.axis_index("s") * _NUM_CORES + lax.axis_index("c")
    trash = batch + wid
    iota = lax.iota(jnp.int32, _L)
    n_my_win = (n_win - 1 - wid) // _NUM_WORKERS + 1

    pltpu.sync_copy(idx_hbm, idx_v.at[pl.ds(0, batch)])

    for q in range(2 * _CHUNK // _L):
      blist_v[q // (_CHUNK // _L), pl.ds((q % (_CHUNK // _L)) * _L, _L)] = (
          jnp.zeros((_L,), jnp.int32) + trash)

    def scan_body(g, m):
      v = idx_v[pl.ds(g * _L, _L)]
      c = lax.shift_right_logical(v, 9)
      mask = (c & (_NUM_WORKERS - 1)) == wid
      key = (lax.shift_left(lax.shift_right_logical(c, 5), 24)
             | lax.shift_left(v & (_WIN - 1), 15)
             | (iota + g * _L))
      pref = plsc.cumsum(mask.astype(jnp.int32))
      pos = jnp.where(mask, m + pref - 1, batch)
      plsc.store_scatter(mk_v, _row_col(pos), key)
      return m + plsc.all_reduce_population_count(mask)[0]

    m_cnt = lax.fori_loop(0, batch // _L, scan_body, 0)
    plsc.store_scatter(mk_v, _row_col(m_cnt + iota),
                       jnp.zeros((_L,), jnp.int32) + (_OSENT << 24))

    def do_window(boff, k, rpos, spos):

      def blk_body(blk, carry):
        base = blk * _WL_CAP

        def wl_body(g, w):
          kk = plsc.load_gather(mk_v, _row_col(base + g * _L + iota))
          mask = lax.shift_right_logical(kk, 24) == k
          pref = plsc.cumsum(mask.astype(jnp.int32))
          pos = jnp.where(mask, w + pref - 1, _WL_CAP)
          plsc.store_scatter(wl_v, _row_col(pos), kk)
          return w + plsc.all_reduce_population_count(mask)[0]

        bcnt = jnp.minimum(m_cnt - base, _WL_CAP)
        w_cnt = lax.fori_loop(0, (bcnt + _L - 1) // _L, wl_body, 0)
        plsc.store_scatter(wl_v, _row_col(w_cnt + iota),
                           jnp.zeros((_L,), jnp.int32) + trash)

        def grp_body(g2, carry):
          rpos, spos = carry
          kk = plsc.load_gather(wl_v, _row_col(g2 * _L + iota))
          colv = lax.shift_right_logical(kk, 15) & (_WIN - 1)
          bv = kk & 32767
          gcnt = jnp.minimum(w_cnt - g2 * _L, _L)
          rowpos = (rpos + iota) & (_RING - 1)
          for j in range(dim):
            jsplat = jnp.zeros((_L,), jnp.int32) + j
            val = plsc.load_gather(win_v, [jsplat + boff, colv])
            plsc.store_scatter(ring_v, [rowpos, jsplat], val)
          plsc.store_scatter(
              blist_v,
              [lax.shift_right_logical(rowpos, 6), rowpos & (_CHUNK - 1)], bv)
          rpos = rpos + gcnt

          def flush(spos):
            cid = (spos // _CHUNK) & 1
            pltpu.sync_copy(
                ring_v.at[pl.ds(pl.multiple_of(cid * _CHUNK, _CHUNK), _CHUNK)],
                out_hbm.at[blist_v.at[cid]])
            return spos + _CHUNK

          spos = lax.cond(rpos - spos >= _CHUNK, flush, lambda s: s, spos)
          return (rpos, spos)

        return lax.fori_loop(0, (w_cnt + _L - 1) // _L, grp_body, carry)

      n_blk = (m_cnt + _WL_CAP - 1) // _WL_CAP
      return lax.fori_loop(0, n_blk, blk_body, (rpos, spos))

    def fetch(k, boff):
      c = wid + k * _NUM_WORKERS
      off = pl.multiple_of(c * _WIN, _WIN)
      return pltpu.async_copy(
          tablet_hbm.at[:, pl.ds(off, _WIN)],
          win_v.at[pl.ds(pl.multiple_of(boff, dim), dim)], wsem)

    @pl.when(n_my_win > 0)
    def _():
      fetch(0, 0)

    def win_body(k, carry):
      boff = (k & 1) * dim

      @pl.when(k + 1 < n_my_win)
      def _():
        fetch(k + 1, dim - boff)

      pltpu.make_async_copy(
          tablet_hbm.at[:, pl.ds(0, _WIN)],
          win_v.at[pl.ds(0, dim)], wsem).wait()
      rpos, spos = do_window(boff, k, *carry)
      return (rpos, spos)

    rpos, spos = lax.fori_loop(0, n_my_win, win_body, (0, 0))

    def tail_fn(carry):
      pltpu.sync_copy(tail_hbm, win_v.at[pl.ds(0, dim), pl.ds(0, 128)])
      rpos, spos = do_window(0, n_my_win, *carry)
      return (rpos, spos)

    rpos, spos = lax.cond(
        wid == tail_owner, tail_fn, lambda carry: carry, (rpos, spos))

    for cid in range(2):
      pltpu.sync_copy(
          ring_v.at[pl.ds(cid * _CHUNK, _CHUNK)],
          out_hbm.at[blist_v.at[cid]])

  return sweep


def kernel(idx, text_embeds):
  vocab, dim = text_embeds.shape
  (batch,) = idx.shape
  tablet = text_embeds.T
  tail_lo = (vocab // _WIN) * _WIN
  tail = jnp.pad(tablet[:, tail_lo:], ((0, 0), (0, 128 - (vocab - tail_lo))))
  out_p = _make_sweep(batch, dim, vocab)(idx.astype(jnp.int32), tablet, tail)
  return out_p[:batch, :dim]


# speedup vs baseline: 1.2641x; 1.0038x over previous
import functools

import jax
import jax.numpy as jnp
from jax import lax
from jax.experimental import pallas as pl
from jax.experimental.pallas import tpu as pltpu
from jax.experimental.pallas import tpu_sc as plsc

_NUM_CORES = 2
_NUM_SUBCORES = 16
_NUM_WORKERS = _NUM_CORES * _NUM_SUBCORES
_L = 16

_WIN = 512
_WL_CAP = 4096
_RING = 128
_CHUNK = 64
_OSENT = 63


def _row_col(pos):
  return [lax.shift_right_logical(pos, 7), pos & 127]


def _make_sweep(batch: int, dim: int, vocab: int):
  n_win = vocab // _WIN
  tail_owner = n_win % _NUM_WORKERS
  wdim = 2 * dim
  out_rows = batch + _NUM_WORKERS
  mesh = plsc.VectorSubcoreMesh(core_axis_name="c", subcore_axis_name="s")

  @functools.partial(
      pl.kernel,
      mesh=mesh,
      out_type=jax.ShapeDtypeStruct((out_rows, wdim), jnp.float32),
      compiler_params=pltpu.CompilerParams(needs_layout_passes=False),
      scratch_types=[
          pltpu.VMEM((batch + _L,), jnp.int32),
          pltpu.VMEM((batch // 128 + 1, 128), jnp.int32),
          pltpu.VMEM((_WL_CAP // 128 + 1, 128), jnp.int32),
          pltpu.VMEM((2 * dim, _WIN), jnp.float32),
          pltpu.VMEM((_RING, wdim), jnp.float32),
          pltpu.VMEM((2, _CHUNK), jnp.int32),
          pltpu.SemaphoreType.DMA,
      ],
  )
  def sweep(idx_hbm, tablet_hbm, tail_hbm, out_hbm,
            idx_v, mk_v, wl_v, win_v, ring_v, blist_v, wsem):
    wid = lax.axis_index("s") * _NUM_CORES + lax.axis_index("c")
    trash = batch + wid
    iota = lax.iota(jnp.int32, _L)
    n_my_win = (n_win - 1 - wid) // _NUM_WORKERS + 1

    def fetch(k, boff):
      c = wid + k * _NUM_WORKERS
      off = pl.multiple_of(c * _WIN, _WIN)
      return pltpu.async_copy(
          tablet_hbm.at[:, pl.ds(off, _WIN)],
          win_v.at[pl.ds(pl.multiple_of(boff, dim), dim)], wsem)

    @pl.when(n_my_win > 0)
    def _():
      fetch(0, 0)

    @pl.when(n_my_win > 1)
    def _():
      fetch(1, dim)

    pltpu.sync_copy(idx_hbm, idx_v.at[pl.ds(0, batch)])

    for q in range(2 * _CHUNK // _L):
      blist_v[q // (_CHUNK // _L), pl.ds((q % (_CHUNK // _L)) * _L, _L)] = (
          jnp.zeros((_L,), jnp.int32) + trash)

    def scan_body(g, m):
      v = idx_v[pl.ds(g * _L, _L)]
      c = lax.shift_right_logical(v, 9)
      mask = (c & (_NUM_WORKERS - 1)) == wid
      key = (lax.shift_left(lax.shift_right_logical(c, 5), 24)
             | lax.shift_left(v & (_WIN - 1), 15)
             | (iota + g * _L))
      pref = plsc.cumsum(mask.astype(jnp.int32))
      pos = jnp.where(mask, m + pref - 1, batch)
      plsc.store_scatter(mk_v, _row_col(pos), key)
      return m + plsc.all_reduce_population_count(mask)[0]

    m_cnt = lax.fori_loop(0, batch // _L, scan_body, 0)
    plsc.store_scatter(mk_v, _row_col(m_cnt + iota),
                       jnp.zeros((_L,), jnp.int32) + (_OSENT << 24))

    def do_window(boff, k, rpos, spos):

      def blk_body(blk, carry):
        base = blk * _WL_CAP

        def wl_body(g, w):
          kk = plsc.load_gather(mk_v, _row_col(base + g * _L + iota))
          mask = lax.shift_right_logical(kk, 24) == k
          pref = plsc.cumsum(mask.astype(jnp.int32))
          pos = jnp.where(mask, w + pref - 1, _WL_CAP)
          plsc.store_scatter(wl_v, _row_col(pos), kk)
          return w + plsc.all_reduce_population_count(mask)[0]

        bcnt = jnp.minimum(m_cnt - base, _WL_CAP)
        w_cnt = lax.fori_loop(0, (bcnt + _L - 1) // _L, wl_body, 0)
        plsc.store_scatter(wl_v, _row_col(w_cnt + iota),
                           jnp.zeros((_L,), jnp.int32) + trash)

        def grp_body(g2, carry):
          rpos, spos = carry
          kk = plsc.load_gather(wl_v, _row_col(g2 * _L + iota))
          colv = lax.shift_right_logical(kk, 15) & (_WIN - 1)
          bv = kk & 32767
          gcnt = jnp.minimum(w_cnt - g2 * _L, _L)
          rowpos = (rpos + iota) & (_RING - 1)
          for j in range(dim):
            jsplat = jnp.zeros((_L,), jnp.int32) + j
            val = plsc.load_gather(win_v, [jsplat + boff, colv])
            plsc.store_scatter(ring_v, [rowpos, jsplat], val)
          plsc.store_scatter(
              blist_v,
              [lax.shift_right_logical(rowpos, 6), rowpos & (_CHUNK - 1)], bv)
          rpos = rpos + gcnt

          def flush(spos):
            cid = (spos // _CHUNK) & 1
            pltpu.sync_copy(
                ring_v.at[pl.ds(pl.multiple_of(cid * _CHUNK, _CHUNK), _CHUNK)],
                out_hbm.at[blist_v.at[cid]])
            return spos + _CHUNK

          spos = lax.cond(rpos - spos >= _CHUNK, flush, lambda s: s, spos)
          return (rpos, spos)

        return lax.fori_loop(0, (w_cnt + _L - 1) // _L, grp_body, carry)

      n_blk = (m_cnt + _WL_CAP - 1) // _WL_CAP
      return lax.fori_loop(0, n_blk, blk_body, (rpos, spos))

    def win_body(k, carry):
      boff = (k & 1) * dim
      pltpu.make_async_copy(
          tablet_hbm.at[:, pl.ds(0, _WIN)],
          win_v.at[pl.ds(0, dim)], wsem).wait()
      rpos, spos = do_window(boff, k, *carry)

      @pl.when(k + 2 < n_my_win)
      def _():
        fetch(k + 2, boff)

      return (rpos, spos)

    rpos, spos = lax.fori_loop(0, n_my_win, win_body, (0, 0))

    def tail_fn(carry):
      pltpu.sync_copy(tail_hbm, win_v.at[pl.ds(0, dim), pl.ds(0, 128)])
      rpos, spos = do_window(0, n_my_win, *carry)
      return (rpos, spos)

    rpos, spos = lax.cond(
        wid == tail_owner, tail_fn, lambda carry: carry, (rpos, spos))

    for cid in range(2):
      pltpu.sync_copy(
          ring_v.at[pl.ds(cid * _CHUNK, _CHUNK)],
          out_hbm.at[blist_v.at[cid]])

  return sweep


def kernel(idx, text_embeds):
  vocab, dim = text_embeds.shape
  (batch,) = idx.shape
  tablet = text_embeds.T
  tail_lo = (vocab // _WIN) * _WIN
  tail = jnp.pad(tablet[:, tail_lo:], ((0, 0), (0, 128 - (vocab - tail_lo))))
  out_p = _make_sweep(batch, dim, vocab)(idx.astype(jnp.int32), tablet, tail)
  return out_p[:batch, :dim]
